# Initial kernel scaffold; baseline (speedup 1.0000x reference)
import jax
import jax.numpy as jnp
from jax.experimental import pallas as pl


def kernel(q_len, k_len, relative_attention_bias):
    raise NotImplementedError("write your pallas kernel here")


# speedup vs baseline: 40.9563x; 40.9563x over previous
import functools
import math

import jax
import jax.numpy as jnp
from jax import lax
from jax.experimental import pallas as pl
from jax.experimental.pallas import tpu as pltpu
from jax.experimental.pallas import tpu_sc as plsc

NUM_HEADS = 16
NUM_BUCKETS = 32
MAX_DISTANCE = 128
Q_LEN = 2048
K_LEN = 2048

NSHIFT = 16
LINE_PAD = 4352
NC = 2
NS = 16


def _line_kernel(table_ref, delta_ref, out_ref):
    shape = (1, NSHIFT, LINE_PAD)
    r = lax.broadcasted_iota(jnp.int32, shape, 1)
    x = lax.broadcasted_iota(jnp.int32, shape, 2)
    rel = x + r - (K_LEN - 1) + delta_ref[0, 0]
    n = -rel
    sign = jnp.where(n > 0, NUM_BUCKETS // 2, 0)
    na = jnp.abs(n)
    half = NUM_BUCKETS // 2
    is_small = na < half
    nc = jnp.maximum(na, 1).astype(jnp.float32)
    log_ratio = jnp.log(nc / half) / math.log(MAX_DISTANCE / half)
    vl = jnp.floor(log_ratio * (NUM_BUCKETS - half)).astype(jnp.int32) + half
    vl = jnp.minimum(vl, NUM_BUCKETS - 1)
    val = jnp.where(is_small, na, vl)
    b = jnp.clip(val + sign, 0, NUM_BUCKETS - 1)
    acc = jnp.zeros(shape, jnp.float32)
    for t in range(NUM_BUCKETS):
        acc = jnp.where(b == t, table_ref[0, 0, t], acc)
    out_ref[...] = acc


def _compute_lines(table, delta):
    table_t3 = jnp.transpose(table).reshape(NUM_HEADS, 1, NUM_BUCKETS)
    return pl.pallas_call(
        _line_kernel,
        grid=(NUM_HEADS,),
        in_specs=[
            pl.BlockSpec((1, 1, NUM_BUCKETS), lambda h: (h, 0, 0)),
            pl.BlockSpec(memory_space=pltpu.SMEM),
        ],
        out_specs=pl.BlockSpec((1, NSHIFT, LINE_PAD), lambda h: (h, 0, 0)),
        out_shape=jax.ShapeDtypeStruct((NUM_HEADS, NSHIFT, LINE_PAD), jnp.float32),
    )(table_t3, delta)


_CHUNK = 16


_LINE_WORDS = NSHIFT * LINE_PAD


def _sc_body(line_hbm, out_hbm, line_v, sem):
    c = lax.axis_index("c")
    s = lax.axis_index("s")
    wid = s * NC + c
    h = wid // 2
    half_sel = wid % 2
    pltpu.sync_copy(
        line_hbm.at[pl.ds(pl.multiple_of(h * _LINE_WORDS, _LINE_WORDS), _LINE_WORDS)],
        line_v,
    )
    i0 = half_sel * (Q_LEN // 2)

    def outer(g, carry):
        base = i0 + g * _CHUNK
        for j in range(_CHUNK):
            i = base + j
            o = (K_LEN - 1) - i
            r = (K_LEN - 1 - j) % NSHIFT
            q = o - r
            src_off = pl.multiple_of(r * LINE_PAD + q, 16)
            dst_off = pl.multiple_of((h * Q_LEN + i) * K_LEN, K_LEN)
            pltpu.make_async_copy(
                line_v.at[pl.ds(src_off, K_LEN)],
                out_hbm.at[pl.ds(dst_off, K_LEN)],
                sem,
            ).start()
        for j in range(_CHUNK):
            pltpu.make_async_copy(
                line_v.at[pl.ds(0, K_LEN)], out_hbm.at[pl.ds(0, K_LEN)], sem
            ).wait()
        return carry

    lax.fori_loop(0, (Q_LEN // 2) // _CHUNK, outer, 0)


_MATERIALIZE_CACHE = []


def _materialize_fn():
    if not _MATERIALIZE_CACHE:
        _MATERIALIZE_CACHE.append(functools.partial(
            pl.kernel,
            out_type=jax.ShapeDtypeStruct((NUM_HEADS * Q_LEN * K_LEN,), jnp.float32),
            mesh=plsc.VectorSubcoreMesh(
                core_axis_name="c", subcore_axis_name="s",
                num_cores=NC, num_subcores=NS,
            ),
            scratch_types=[
                pltpu.VMEM((_LINE_WORDS,), jnp.float32),
                pltpu.SemaphoreType.DMA,
            ],
        )(_sc_body))
    return _MATERIALIZE_CACHE[0]


def kernel(q_len, k_len, relative_attention_bias):
    q_res = jnp.asarray(q_len, jnp.int32) - Q_LEN
    k_res = jnp.asarray(k_len, jnp.int32) - K_LEN
    delta = (k_res - q_res).reshape(1, 1)
    lines = _compute_lines(relative_attention_bias, delta)
    out = _materialize_fn()(lines.reshape(-1))
    return out.reshape(1, NUM_HEADS, Q_LEN, K_LEN)


# speedup vs baseline: 112.2882x; 2.7417x over previous
import functools
import math

import jax
import jax.numpy as jnp
from jax import lax
from jax.experimental import pallas as pl
from jax.experimental.pallas import tpu as pltpu
from jax.experimental.pallas import tpu_sc as plsc

NUM_HEADS = 16
NUM_BUCKETS = 32
MAX_DISTANCE = 128
Q_LEN = 2048
K_LEN = 2048

LINE_PAD = 4224
VAR_W = 4096
NC = 2
NS = 16
HEADS_PER_SC = NUM_HEADS // NC
GROUPS = Q_LEN // 8
K_PER_TEC = GROUPS // NS


def _line_kernel(table_ref, delta_ref, out_ref):
    shape = (1, 1, LINE_PAD)
    x = lax.broadcasted_iota(jnp.int32, shape, 2)
    rel = x - (K_LEN - 1) + delta_ref[0, 0]
    n = -rel
    sign = jnp.where(n > 0, NUM_BUCKETS // 2, 0)
    na = jnp.abs(n)
    half = NUM_BUCKETS // 2
    is_small = na < half
    nc = jnp.maximum(na, 1).astype(jnp.float32)
    log_ratio = jnp.log(nc / half) / math.log(MAX_DISTANCE / half)
    vl = jnp.floor(log_ratio * (NUM_BUCKETS - half)).astype(jnp.int32) + half
    vl = jnp.minimum(vl, NUM_BUCKETS - 1)
    val = jnp.where(is_small, na, vl)
    b = jnp.clip(val + sign, 0, NUM_BUCKETS - 1)
    line = jnp.zeros(shape, jnp.float32)
    for t in range(NUM_BUCKETS):
        line = jnp.where(b == t, table_ref[0, 0, t], line)
    for t in range(NS):
        for r8 in range(8):
            s = 127 - 8 * t - r8
            out_ref[0, t, r8, :] = lax.slice(line, (0, 0, s), (1, 1, s + VAR_W))[0, 0]


def _compute_linev8(table, delta):
    table_t3 = jnp.transpose(table).reshape(NUM_HEADS, 1, NUM_BUCKETS)
    return pl.pallas_call(
        _line_kernel,
        grid=(NUM_HEADS,),
        in_specs=[
            pl.BlockSpec((1, 1, NUM_BUCKETS), lambda h: (h, 0, 0)),
            pl.BlockSpec(memory_space=pltpu.SMEM),
        ],
        out_specs=pl.BlockSpec((1, NS, 8, VAR_W), lambda h: (h, 0, 0, 0)),
        out_shape=jax.ShapeDtypeStruct((NUM_HEADS, NS, 8, VAR_W), jnp.float32),
    )(table_t3, delta)


def _sc_body(linev8_hbm, out_hbm, var_a, var_b, load_sem, write_sem):
    c = lax.axis_index("c")
    t = lax.axis_index("s")
    bufs = (var_a, var_b)

    def load(hh, buf):
        h = c * HEADS_PER_SC + hh
        return pltpu.make_async_copy(linev8_hbm.at[h, t], buf, load_sem)

    load(0, bufs[0]).start()
    for hh in range(HEADS_PER_SC):
        buf = bufs[hh % 2]
        load(hh, buf).wait()
        if hh + 1 < HEADS_PER_SC:
            load(hh + 1, bufs[(hh + 1) % 2]).start()
        h = c * HEADS_PER_SC + hh
        for k in range(K_PER_TEC):
            p0 = (K_PER_TEC - 1 - k) * 128
            i8 = pl.multiple_of(128 * k + 8 * t, 8)
            pltpu.make_async_copy(
                buf.at[:, pl.ds(p0, K_LEN)],
                out_hbm.at[0, h, pl.ds(i8, 8), :],
                write_sem,
            ).start()
        for k in range(K_PER_TEC):
            pltpu.make_async_copy(
                buf.at[:, pl.ds(0, K_LEN)],
                out_hbm.at[0, h, pl.ds(0, 8), :],
                write_sem,
            ).wait()


_MATERIALIZE_CACHE = []


def _materialize_fn():
    if not _MATERIALIZE_CACHE:
        _MATERIALIZE_CACHE.append(functools.partial(
            pl.kernel,
            out_type=jax.ShapeDtypeStruct((1, NUM_HEADS, Q_LEN, K_LEN), jnp.float32),
            mesh=plsc.VectorSubcoreMesh(
                core_axis_name="c", subcore_axis_name="s",
                num_cores=NC, num_subcores=NS,
            ),
            scratch_types=[
                pltpu.VMEM((8, VAR_W), jnp.float32),
                pltpu.VMEM((8, VAR_W), jnp.float32),
                pltpu.SemaphoreType.DMA,
                pltpu.SemaphoreType.DMA,
            ],
        )(_sc_body))
    return _MATERIALIZE_CACHE[0]


def kernel(q_len, k_len, relative_attention_bias):
    q_res = jnp.asarray(q_len, jnp.int32) - Q_LEN
    k_res = jnp.asarray(k_len, jnp.int32) - K_LEN
    delta = (k_res - q_res).reshape(1, 1)
    linev8 = _compute_linev8(relative_attention_bias, delta)
    return _materialize_fn()(linev8)


# speedup vs baseline: 112.7906x; 1.0045x over previous
import functools
import math

import jax
import jax.numpy as jnp
from jax import lax
from jax.experimental import pallas as pl
from jax.experimental.pallas import tpu as pltpu
from jax.experimental.pallas import tpu_sc as plsc

NUM_HEADS = 16
NUM_BUCKETS = 32
MAX_DISTANCE = 128
Q_LEN = 2048
K_LEN = 2048

LINE_PAD = 4224
VAR_W = 3968
NC = 2
NS = 16
HEADS_PER_SC = NUM_HEADS // NC
GROUPS = Q_LEN // 8
K_PER_TEC = GROUPS // NS


def _line_kernel(table_ref, delta_ref, out_ref):
    shape = (1, 1, LINE_PAD)
    x = lax.broadcasted_iota(jnp.int32, shape, 2)
    rel = x - (K_LEN - 1) + delta_ref[0, 0]
    n = -rel
    sign = jnp.where(n > 0, NUM_BUCKETS // 2, 0)
    na = jnp.abs(n)
    half = NUM_BUCKETS // 2
    is_small = na < half
    nc = jnp.maximum(na, 1).astype(jnp.float32)
    log_ratio = jnp.log(nc / half) / math.log(MAX_DISTANCE / half)
    vl = jnp.floor(log_ratio * (NUM_BUCKETS - half)).astype(jnp.int32) + half
    vl = jnp.minimum(vl, NUM_BUCKETS - 1)
    val = jnp.where(is_small, na, vl)
    b = jnp.clip(val + sign, 0, NUM_BUCKETS - 1)
    line = jnp.zeros(shape, jnp.float32)
    for t in range(NUM_BUCKETS):
        line = jnp.where(b == t, table_ref[0, 0, t], line)
    for t in range(NS):
        for r8 in range(8):
            s = 127 - 8 * t - r8
            out_ref[0, t, r8, :] = lax.slice(line, (0, 0, s), (1, 1, s + VAR_W))[0, 0]


def _compute_linev8(table, delta):
    table_t3 = jnp.transpose(table).reshape(NUM_HEADS, 1, NUM_BUCKETS)
    return pl.pallas_call(
        _line_kernel,
        grid=(NUM_HEADS,),
        in_specs=[
            pl.BlockSpec((1, 1, NUM_BUCKETS), lambda h: (h, 0, 0)),
            pl.BlockSpec(memory_space=pltpu.SMEM),
        ],
        out_specs=pl.BlockSpec((1, NS, 8, VAR_W), lambda h: (h, 0, 0, 0)),
        out_shape=jax.ShapeDtypeStruct((NUM_HEADS, NS, 8, VAR_W), jnp.float32),
    )(table_t3, delta)


def _sc_body(linev8_hbm, out_hbm, var_a, var_b, load_sem, write_sem):
    c = lax.axis_index("c")
    t = lax.axis_index("s")
    bufs = (var_a, var_b)

    def load(hh, buf):
        h = c * HEADS_PER_SC + hh
        return pltpu.make_async_copy(linev8_hbm.at[h, t], buf, load_sem)

    def drain_writes():
        for _ in range(K_PER_TEC):
            pltpu.make_async_copy(
                bufs[0].at[:, pl.ds(0, K_LEN)],
                out_hbm.at[0, 0, pl.ds(0, 8), :],
                write_sem,
            ).wait()

    load(0, bufs[0]).start()
    for hh in range(HEADS_PER_SC):
        buf = bufs[hh % 2]
        load(hh, buf).wait()
        h = c * HEADS_PER_SC + hh
        for k in range(K_PER_TEC):
            p0 = (K_PER_TEC - 1 - k) * 128
            i8 = pl.multiple_of(128 * k + 8 * t, 8)
            pltpu.make_async_copy(
                buf.at[:, pl.ds(p0, K_LEN)],
                out_hbm.at[0, h, pl.ds(i8, 8), :],
                write_sem,
            ).start()
        if hh >= 1:
            drain_writes()
        if hh + 1 < HEADS_PER_SC:
            load(hh + 1, bufs[(hh + 1) % 2]).start()
    drain_writes()


_MATERIALIZE_CACHE = []


def _materialize_fn():
    if not _MATERIALIZE_CACHE:
        _MATERIALIZE_CACHE.append(functools.partial(
            pl.kernel,
            out_type=jax.ShapeDtypeStruct((1, NUM_HEADS, Q_LEN, K_LEN), jnp.float32),
            mesh=plsc.VectorSubcoreMesh(
                core_axis_name="c", subcore_axis_name="s",
                num_cores=NC, num_subcores=NS,
            ),
            scratch_types=[
                pltpu.VMEM((8, VAR_W), jnp.float32),
                pltpu.VMEM((8, VAR_W), jnp.float32),
                pltpu.SemaphoreType.DMA,
                pltpu.SemaphoreType.DMA,
            ],
        )(_sc_body))
    return _MATERIALIZE_CACHE[0]


def kernel(q_len, k_len, relative_attention_bias):
    q_res = jnp.asarray(q_len, jnp.int32) - Q_LEN
    k_res = jnp.asarray(k_len, jnp.int32) - K_LEN
    delta = (k_res - q_res).reshape(1, 1)
    linev8 = _compute_linev8(relative_attention_bias, delta)
    return _materialize_fn()(linev8)
